# baseline (speedup 1.0000x reference)
import functools

import jax
import jax.numpy as jnp
from jax import lax
from jax.experimental import pallas as pl
from jax.experimental.pallas import tpu as pltpu
from jax.experimental.pallas import tpu_sc as plsc

_B = 16384
_V = 64
_D = 128
_NC = 2
_NS = 16
_NW = _NC * _NS
_BW = _B // _NW
_L = 16
_CHUNKS = _BW // _L
_RPW = _V // _NS
_DC = _D // _L


_sc_mesh = plsc.VectorSubcoreMesh(core_axis_name="c", subcore_axis_name="s")


@functools.partial(
    pl.kernel,
    mesh=_sc_mesh,
    out_type=[
        jax.ShapeDtypeStruct((_B,), jnp.float32),
        jax.ShapeDtypeStruct((_B,), jnp.float32),
    ],
    scratch_types=[
        pltpu.VMEM((_RPW * _D,), jnp.float32),
        pltpu.VMEM((2 * _D,), jnp.float32),
        pltpu.VMEM((2,), jnp.float32),
        pltpu.VMEM((_L,), jnp.float32),
        pltpu.VMEM((2 * _V,), jnp.float32),
        pltpu.VMEM((_BW,), jnp.int32),
        pltpu.VMEM((_BW,), jnp.float32),
        pltpu.VMEM((_BW,), jnp.float32),
        pltpu.VMEM_SHARED((2 * _V,), jnp.float32),
        pltpu.SemaphoreType.DMA,
    ],
    compiler_params=pltpu.CompilerParams(needs_layout_passes=False),
)
def _encoder_sc(table_hbm, wf_hbm, b_hbm, idx_hbm, g_hbm, bt_hbm,
                t4_v, wf_v, b_v, piece_v, ft_v, idx_v, g_v, bt_v, ft_sh, sem):
    cid = lax.axis_index("c")
    sid = lax.axis_index("s")
    wid = sid * _NC + cid
    base = wid * _BW

    cpy_t = pltpu.async_copy(
        table_hbm.at[pl.ds(_RPW * _D * sid, _RPW * _D)], t4_v, sem)
    cpy_w = pltpu.async_copy(wf_hbm, wf_v, sem)
    cpy_b = pltpu.async_copy(b_hbm, b_v, sem)
    cpy_i = pltpu.async_copy(idx_hbm.at[pl.ds(base, _BW)], idx_v, sem)
    cpy_t.wait()
    cpy_w.wait()
    cpy_b.wait()

    lanes = lax.iota(jnp.int32, _L)
    wg = []
    wb = []
    for c in range(_DC):
        flat = (lanes + (c * _L)) * 2
        wg.append(plsc.load_gather(wf_v, [flat]))
        wb.append(plsc.load_gather(wf_v, [flat + 1]))

    bias = plsc.load_gather(b_v, [lanes % 2])
    piece = jnp.zeros((_L,), jnp.float32)
    for vv in range(_RPW):
        acc_g = jnp.zeros((_L,), jnp.float32)
        acc_b = jnp.zeros((_L,), jnp.float32)
        for c in range(_DC):
            t = t4_v[pl.ds(vv * _D + c * _L, _L)]
            acc_g = acc_g + t * wg[c]
            acc_b = acc_b + t * wb[c]
        piece = jnp.where(lanes == 2 * vv, jnp.sum(acc_g), piece)
        piece = jnp.where(lanes == 2 * vv + 1, jnp.sum(acc_b), piece)
    piece_v[...] = piece + bias

    pltpu.sync_copy(piece_v.at[pl.ds(0, 2 * _RPW)],
                    ft_sh.at[pl.ds(2 * _RPW * sid, 2 * _RPW)])
    plsc.subcore_barrier()
    pltpu.sync_copy(ft_sh, ft_v)

    cpy_i.wait()
    for i in range(_CHUNKS):
        sl = pl.ds(i * _L, _L)
        ix2 = idx_v[sl] * 2
        g_v[sl] = plsc.load_gather(ft_v, [ix2])
        bt_v[sl] = plsc.load_gather(ft_v, [ix2 + 1])
    cpy_g = pltpu.async_copy(g_v, g_hbm.at[pl.ds(base, _BW)], sem)
    cpy_bt = pltpu.async_copy(bt_v, bt_hbm.at[pl.ds(base, _BW)], sem)
    cpy_g.wait()
    cpy_bt.wait()


def kernel(effect_id, table, W, b):
    tf = table.reshape(_V * _D)
    wf = W.reshape(2 * _D)
    idx = effect_id.reshape(_B)
    gamma, beta = _encoder_sc(tf, wf, b, idx)
    return gamma.reshape(_B, 1, 1), beta.reshape(_B, 1, 1)


# speedup vs baseline: 1.0248x; 1.0248x over previous
import functools

import jax
import jax.numpy as jnp
from jax import lax
from jax.experimental import pallas as pl
from jax.experimental.pallas import tpu as pltpu
from jax.experimental.pallas import tpu_sc as plsc

_B = 16384
_V = 64
_D = 128
_NC = 2
_NS = 16
_NW = _NC * _NS
_BW = _B // _NW
_L = 16
_CHUNKS = _BW // _L
_RPW = _V // _NS
_DC = _D // _L


_sc_mesh = plsc.VectorSubcoreMesh(core_axis_name="c", subcore_axis_name="s")


@functools.partial(
    pl.kernel,
    mesh=_sc_mesh,
    out_type=[
        jax.ShapeDtypeStruct((_B,), jnp.float32),
        jax.ShapeDtypeStruct((_B,), jnp.float32),
    ],
    scratch_types=[
        pltpu.VMEM((_RPW * _D,), jnp.float32),
        pltpu.VMEM((2 * _D,), jnp.float32),
        pltpu.VMEM((2,), jnp.float32),
        pltpu.VMEM((_L,), jnp.float32),
        pltpu.VMEM((2 * _V,), jnp.float32),
        pltpu.VMEM((_BW,), jnp.int32),
        pltpu.VMEM((_BW,), jnp.float32),
        pltpu.VMEM((_BW,), jnp.float32),
        pltpu.VMEM_SHARED((2 * _V,), jnp.float32),
        pltpu.SemaphoreType.DMA,
    ],
    compiler_params=pltpu.CompilerParams(needs_layout_passes=False),
)
def _encoder_sc(table_hbm, wf_hbm, b_hbm, idx_hbm, g_hbm, bt_hbm,
                t4_v, wf_v, b_v, piece_v, ft_v, idx_v, g_v, bt_v, ft_sh, sem):
    cid = lax.axis_index("c")
    sid = lax.axis_index("s")
    wid = sid * _NC + cid
    base = wid * _BW

    cpy_t = pltpu.async_copy(
        table_hbm.at[pl.ds(_RPW * _D * sid, _RPW * _D)], t4_v, sem)
    cpy_w = pltpu.async_copy(wf_hbm, wf_v, sem)
    cpy_b = pltpu.async_copy(b_hbm, b_v, sem)
    cpy_i = pltpu.async_copy(idx_hbm.at[pl.ds(base, _BW)], idx_v, sem)
    cpy_t.wait()
    cpy_w.wait()
    cpy_b.wait()

    lanes = lax.iota(jnp.int32, _L)

    def film_body(c, accs):
        flat = (lanes + c * _L) * 2
        wgc = plsc.load_gather(wf_v, [flat])
        wbc = plsc.load_gather(wf_v, [flat + 1])
        out = []
        for vv in range(_RPW):
            t = t4_v[pl.ds(vv * _D + c * _L, _L)]
            out.append(accs[2 * vv] + t * wgc)
            out.append(accs[2 * vv + 1] + t * wbc)
        return tuple(out)

    accs = lax.fori_loop(
        0, _DC, film_body,
        tuple(jnp.zeros((_L,), jnp.float32) for _ in range(2 * _RPW)))

    bias = plsc.load_gather(b_v, [lanes % 2])
    piece = jnp.zeros((_L,), jnp.float32)
    for vv in range(_RPW):
        piece = jnp.where(lanes == 2 * vv, jnp.sum(accs[2 * vv]), piece)
        piece = jnp.where(lanes == 2 * vv + 1, jnp.sum(accs[2 * vv + 1]), piece)
    piece_v[...] = piece + bias

    pltpu.sync_copy(piece_v.at[pl.ds(0, 2 * _RPW)],
                    ft_sh.at[pl.ds(2 * _RPW * sid, 2 * _RPW)])
    plsc.subcore_barrier()
    pltpu.sync_copy(ft_sh, ft_v)

    cpy_i.wait()

    def gather_body(i, carry):
        sl = pl.ds(i * _L, _L)
        ix2 = idx_v[sl] * 2
        g_v[sl] = plsc.load_gather(ft_v, [ix2])
        bt_v[sl] = plsc.load_gather(ft_v, [ix2 + 1])
        return carry

    lax.fori_loop(0, _CHUNKS, gather_body, 0)
    cpy_g = pltpu.async_copy(g_v, g_hbm.at[pl.ds(base, _BW)], sem)
    cpy_bt = pltpu.async_copy(bt_v, bt_hbm.at[pl.ds(base, _BW)], sem)
    cpy_g.wait()
    cpy_bt.wait()


def kernel(effect_id, table, W, b):
    tf = table.reshape(_V * _D)
    wf = W.reshape(2 * _D)
    idx = effect_id.reshape(_B)
    gamma, beta = _encoder_sc(tf, wf, b, idx)
    return gamma.reshape(_B, 1, 1), beta.reshape(_B, 1, 1)


# speedup vs baseline: 1.0347x; 1.0096x over previous
import functools

import jax
import jax.numpy as jnp
from jax import lax
from jax.experimental import pallas as pl
from jax.experimental.pallas import tpu as pltpu
from jax.experimental.pallas import tpu_sc as plsc

_B = 16384
_V = 64
_D = 128
_NC = 2
_NS = 16
_NW = _NC * _NS
_BW = _B // _NW
_L = 16
_CHUNKS = _BW // _L
_RPW = _V // _NS
_DC = _D // _L


_sc_mesh = plsc.VectorSubcoreMesh(core_axis_name="c", subcore_axis_name="s")


@functools.partial(
    pl.kernel,
    mesh=_sc_mesh,
    out_type=[
        jax.ShapeDtypeStruct((_B,), jnp.float32),
        jax.ShapeDtypeStruct((_B,), jnp.float32),
    ],
    scratch_types=[
        pltpu.VMEM((_RPW * _D,), jnp.float32),
        pltpu.VMEM((2 * _D,), jnp.float32),
        pltpu.VMEM((2,), jnp.float32),
        pltpu.VMEM((_L,), jnp.float32),
        pltpu.VMEM((2 * _V,), jnp.float32),
        pltpu.VMEM((_BW,), jnp.int32),
        pltpu.VMEM((_BW,), jnp.float32),
        pltpu.VMEM((_BW,), jnp.float32),
        pltpu.VMEM_SHARED((2 * _V,), jnp.float32),
        pltpu.SemaphoreType.DMA,
    ],
    compiler_params=pltpu.CompilerParams(needs_layout_passes=False),
)
def _encoder_sc(table_hbm, wf_hbm, b_hbm, idx_hbm, g_hbm, bt_hbm,
                t4_v, wf_v, b_v, piece_v, ft_v, idx_v, g_v, bt_v, ft_sh, sem):
    cid = lax.axis_index("c")
    sid = lax.axis_index("s")
    wid = sid * _NC + cid
    base = wid * _BW

    cpy_t = [
        pltpu.async_copy(
            table_hbm.at[pl.ds(_RPW * _D * sid, _RPW * _D)], t4_v, sem)
    ]
    cpy_w = pltpu.async_copy(wf_hbm, wf_v, sem)
    cpy_b = pltpu.async_copy(b_hbm, b_v, sem)
    cpy_i = pltpu.async_copy(idx_hbm.at[pl.ds(base, _BW)], idx_v, sem)
    for cp in cpy_t:
        cp.wait()
    cpy_w.wait()
    cpy_b.wait()

    lanes = lax.iota(jnp.int32, _L)

    def film_body(c, accs):
        flat = (lanes + c * _L) * 2
        wgc = plsc.load_gather(wf_v, [flat])
        wbc = plsc.load_gather(wf_v, [flat + 1])
        out = []
        for vv in range(_RPW):
            t = t4_v[pl.ds(vv * _D + c * _L, _L)]
            out.append(accs[2 * vv] + t * wgc)
            out.append(accs[2 * vv + 1] + t * wbc)
        return tuple(out)

    accs = lax.fori_loop(
        0, _DC, film_body,
        tuple(jnp.zeros((_L,), jnp.float32) for _ in range(2 * _RPW)))

    bias = plsc.load_gather(b_v, [lanes % 2])
    piece = jnp.zeros((_L,), jnp.float32)
    for vv in range(_RPW):
        piece = jnp.where(lanes == 2 * vv, jnp.sum(accs[2 * vv]), piece)
        piece = jnp.where(lanes == 2 * vv + 1, jnp.sum(accs[2 * vv + 1]), piece)
    piece_v[...] = piece + bias

    pltpu.sync_copy(piece_v.at[pl.ds(0, 2 * _RPW)],
                    ft_sh.at[pl.ds(2 * _RPW * sid, 2 * _RPW)])
    plsc.subcore_barrier()
    pltpu.sync_copy(ft_sh, ft_v)

    cpy_i.wait()

    @plsc.parallel_loop(0, _BW, step=_L, unroll=4)
    def _(off):
        sl = pl.ds(off, _L)
        ix2 = idx_v[sl] * 2
        g_v[sl] = plsc.load_gather(ft_v, [ix2])
        bt_v[sl] = plsc.load_gather(ft_v, [ix2 + 1])
    cpy_g = pltpu.async_copy(g_v, g_hbm.at[pl.ds(base, _BW)], sem)
    cpy_bt = pltpu.async_copy(bt_v, bt_hbm.at[pl.ds(base, _BW)], sem)
    cpy_g.wait()
    cpy_bt.wait()


def kernel(effect_id, table, W, b):
    tf = table.reshape(_V * _D)
    wf = W.reshape(2 * _D)
    idx = effect_id.reshape(_B)
    gamma, beta = _encoder_sc(tf, wf, b, idx)
    return gamma.reshape(_B, 1, 1), beta.reshape(_B, 1, 1)
